# Initial kernel scaffold; baseline (speedup 1.0000x reference)
import jax
import jax.numpy as jnp
from jax.experimental import pallas as pl


def kernel(z, edge_index):
    raise NotImplementedError("write your pallas kernel here")


# speedup vs baseline: 1.0395x; 1.0395x over previous
import functools

import jax
import jax.numpy as jnp
from jax import lax
from jax.experimental import pallas as pl
from jax.experimental.pallas import tpu as pltpu
from jax.experimental.pallas import tpu_sc as plsc

B = 320000
D = 128
NC, NS, L = 2, 16, 16
NW = NC * NS
E_W = B // NW
E = 80
N_CHUNK = E_W // E


def _edge_dot_kernel(z_hbm, src_hbm, dst_hbm, out_hbm,
                     sidx_v, didx_v, srows_v, drows_v, out_v, sem):
    wid = lax.axis_index("s") * NC + lax.axis_index("c")
    lane = lax.iota(jnp.int32, L)

    def chunk_body(c, _):
        base = wid * E_W + c * E
        pltpu.sync_copy(src_hbm.at[pl.ds(base, E)], sidx_v)
        pltpu.sync_copy(dst_hbm.at[pl.ds(base, E)], didx_v)
        pltpu.async_copy(z_hbm.at[sidx_v], srows_v, sem).wait()
        pltpu.async_copy(z_hbm.at[didx_v], drows_v, sem).wait()
        for g in range(E // L):
            row = jnp.full((L,), g * L, jnp.int32) + lane

            def f_body(f, acc):
                col = jnp.full((L,), f, jnp.int32)
                s = plsc.load_gather(srows_v, [row, col])
                d = plsc.load_gather(drows_v, [row, col])
                return acc + s * d

            acc = lax.fori_loop(0, D, f_body, jnp.zeros((L,), jnp.float32))
            out_v[pl.ds(g * L, L)] = acc
        pltpu.sync_copy(out_v, out_hbm.at[pl.ds(base, E)])
        return 0

    lax.fori_loop(0, N_CHUNK, chunk_body, 0)


@jax.jit
def kernel(z, edge_index):
    src = edge_index[0].astype(jnp.int32)
    dst = edge_index[1].astype(jnp.int32)
    mesh = plsc.VectorSubcoreMesh(core_axis_name="c", subcore_axis_name="s")
    run = functools.partial(
        pl.kernel,
        mesh=mesh,
        out_type=jax.ShapeDtypeStruct((B,), jnp.float32),
        compiler_params=pltpu.CompilerParams(
            use_tc_tiling_on_sc=False, needs_layout_passes=False),
        scratch_types=[
            pltpu.VMEM((E,), jnp.int32),
            pltpu.VMEM((E,), jnp.int32),
            pltpu.VMEM((E, D), jnp.float32),
            pltpu.VMEM((E, D), jnp.float32),
            pltpu.VMEM((E,), jnp.float32),
            pltpu.SemaphoreType.DMA,
        ],
    )(_edge_dot_kernel)
    return run(z, src, dst)


# speedup vs baseline: 9.1089x; 8.7624x over previous
import functools

import jax
import jax.numpy as jnp
from jax import lax
from jax.experimental import pallas as pl
from jax.experimental.pallas import tpu as pltpu
from jax.experimental.pallas import tpu_sc as plsc

B = 320000
D = 128
N = 10000
NC, NS, L = 2, 16, 16
F = D // NS
E_SC = B // NC
C = 4000
N_CHUNK = E_SC // C
N_PAIR = N_CHUNK // 2
ACC_W = E_SC // NS


def _edge_dot_kernel(zt_hbm, src_hbm, dst_hbm, out_hbm,
                     zloc, sidx_a, didx_a, sidx_b, didx_b,
                     part_a, part_b, ramp_a, ramp_b, acc_sh,
                     sem_z, sem_ia, sem_ib, sem_aa, sem_ab):
    c = lax.axis_index("c")
    s = lax.axis_index("s")
    ebase = c * E_SC
    lane = lax.iota(jnp.int32, L)
    zero = jnp.zeros((L,), jnp.float32)

    pltpu.async_copy(zt_hbm.at[pl.ds(s * (F * N), F * N)], zloc, sem_z)

    @plsc.parallel_loop(0, C, L)
    def _(i):
        part_a[pl.ds(i, L)] = zero

    for off in range(0, ACC_W, C):
        w = min(C, ACC_W - off)
        pltpu.sync_copy(part_a.at[pl.ds(0, w)],
                        acc_sh.at[pl.ds(s * ACC_W + off, w)])
    pltpu.make_async_copy(zt_hbm.at[pl.ds(0, F * N)], zloc, sem_z).wait()
    plsc.subcore_barrier()

    def issue_idx(k, sidx, didx, sem):
        pltpu.async_copy(src_hbm.at[pl.ds(ebase + k * C, C)], sidx, sem)
        pltpu.async_copy(dst_hbm.at[pl.ds(ebase + k * C, C)], didx, sem)

    def wait_idx(sidx, didx, sem):
        pltpu.make_async_copy(src_hbm.at[pl.ds(0, C)], sidx, sem).wait()
        pltpu.make_async_copy(src_hbm.at[pl.ds(0, C)], didx, sem).wait()

    def drain_add(part, sem):
        pltpu.make_async_copy(part, acc_sh.at[pl.ds(0, C)], sem).wait()

    def compute(k, sidx, didx, part, ramp, sem_a):
        kbase = jnp.full((L,), k * C, jnp.int32) + lane

        @plsc.parallel_loop(0, C, L, unroll=2)
        def _(g):
            sn = sidx[pl.ds(g, L)]
            dn = didx[pl.ds(g, L)]
            ramp[pl.ds(g, L)] = kbase + g
            acc = zero
            for j in range(F):
                off = jnp.full((L,), j * N, jnp.int32)
                acc += (plsc.load_gather(zloc, [sn + off]) *
                        plsc.load_gather(zloc, [dn + off]))
            part[pl.ds(g, L)] = acc

        pltpu.async_copy(part, acc_sh.at[ramp], sem_a, add=True)

    issue_idx(0, sidx_a, didx_a, sem_ia)

    def pair_body(p, _):
        ka = 2 * p
        issue_idx(ka + 1, sidx_b, didx_b, sem_ib)
        wait_idx(sidx_a, didx_a, sem_ia)

        @pl.when(p > 0)
        def _():
            drain_add(part_a, sem_aa)

        compute(ka, sidx_a, didx_a, part_a, ramp_a, sem_aa)

        @pl.when(p < N_PAIR - 1)
        def _():
            issue_idx(ka + 2, sidx_a, didx_a, sem_ia)

        wait_idx(sidx_b, didx_b, sem_ib)

        @pl.when(p > 0)
        def _():
            drain_add(part_b, sem_ab)

        compute(ka + 1, sidx_b, didx_b, part_b, ramp_b, sem_ab)
        return 0

    lax.fori_loop(0, N_PAIR, pair_body, 0)
    drain_add(part_a, sem_aa)
    drain_add(part_b, sem_ab)
    plsc.subcore_barrier()

    pltpu.sync_copy(acc_sh.at[pl.ds(s * ACC_W, ACC_W)],
                    out_hbm.at[pl.ds(ebase + s * ACC_W, ACC_W)])


@jax.jit
def kernel(z, edge_index):
    zt = z.T.reshape(D * N)
    src = edge_index[0].astype(jnp.int32)
    dst = edge_index[1].astype(jnp.int32)
    mesh = plsc.VectorSubcoreMesh(core_axis_name="c", subcore_axis_name="s")
    run = functools.partial(
        pl.kernel,
        mesh=mesh,
        out_type=jax.ShapeDtypeStruct((B,), jnp.float32),
        compiler_params=pltpu.CompilerParams(
            use_tc_tiling_on_sc=False, needs_layout_passes=False),
        scratch_types=[
            pltpu.VMEM((F * N,), jnp.float32),
            pltpu.VMEM((C,), jnp.int32),
            pltpu.VMEM((C,), jnp.int32),
            pltpu.VMEM((C,), jnp.int32),
            pltpu.VMEM((C,), jnp.int32),
            pltpu.VMEM((C,), jnp.float32),
            pltpu.VMEM((C,), jnp.float32),
            pltpu.VMEM((C,), jnp.int32),
            pltpu.VMEM((C,), jnp.int32),
            pltpu.VMEM_SHARED((E_SC,), jnp.float32),
            pltpu.SemaphoreType.DMA,
            pltpu.SemaphoreType.DMA,
            pltpu.SemaphoreType.DMA,
            pltpu.SemaphoreType.DMA,
            pltpu.SemaphoreType.DMA,
        ],
    )(_edge_dot_kernel)
    return run(zt, src, dst)


# speedup vs baseline: 11.4935x; 1.2618x over previous
import functools

import jax
import jax.numpy as jnp
from jax import lax
from jax.experimental import pallas as pl
from jax.experimental.pallas import tpu as pltpu
from jax.experimental.pallas import tpu_sc as plsc

B = 320000
D = 128
N = 10000
W = 64
NC, NS, L = 2, 16, 16
NW = NC * NS
FW = W // NS

NITER = 80
E = 64
Y_W = NITER * E
Y = Y_W * NW
CF = 976
X_SC = NITER * CF
ACC_W = X_SC // NS
N_STAGE = N // NS


def _edge_dot_kernel(zp_hbm, zpt_hbm, src_hbm, dst_hbm, out_hbm,
                     zsh, acc_sh, zcol, sidx_v, didx_v,
                     sbuf_a, dbuf_a, sbuf_b, dbuf_b, out_v,
                     fsidx_a, fdidx_a, fsidx_b, fdidx_b,
                     part_a, part_b, ramp_a, ramp_b,
                     sem_z, sem_a, sem_b, sem_fa, sem_fb,
                     sem_pa, sem_pb):
    c = lax.axis_index("c")
    s = lax.axis_index("s")
    wid = s * NC + c
    sbase = wid * Y_W
    fbase = Y + c * X_SC
    lane = lax.iota(jnp.int32, L)
    m15 = lane == 15
    zero = jnp.zeros((L,), jnp.float32)

    pltpu.async_copy(src_hbm.at[pl.ds(sbase, Y_W)], sidx_v, sem_z)
    pltpu.async_copy(dst_hbm.at[pl.ds(sbase, Y_W)], didx_v, sem_z)
    pltpu.async_copy(zpt_hbm.at[pl.ds(s * (FW * N), FW * N)], zcol, sem_z)
    pltpu.sync_copy(zp_hbm.at[pl.ds(s * N_STAGE, N_STAGE)],
                    zsh.at[pl.ds(s * N_STAGE, N_STAGE)])

    @plsc.parallel_loop(0, CF, L)
    def _(i):
        part_a[pl.ds(i, L)] = zero

    for off in range(0, ACC_W, CF):
        w = min(CF, ACC_W - off)
        pltpu.sync_copy(part_a.at[pl.ds(0, w)],
                        acc_sh.at[pl.ds(s * ACC_W + off, w)])
    pltpu.make_async_copy(src_hbm.at[pl.ds(0, Y_W)], sidx_v, sem_z).wait()
    pltpu.make_async_copy(src_hbm.at[pl.ds(0, Y_W)], didx_v, sem_z).wait()
    pltpu.make_async_copy(zpt_hbm.at[pl.ds(0, FW * N)], zcol, sem_z).wait()
    plsc.subcore_barrier()

    def issue(k, sbuf, dbuf, sem):
        pltpu.async_copy(zsh.at[sidx_v.at[pl.ds(k * E, E)]], sbuf, sem)
        pltpu.async_copy(zsh.at[didx_v.at[pl.ds(k * E, E)]], dbuf, sem)

    def wait(sbuf, dbuf, sem):
        pltpu.make_async_copy(zsh.at[pl.ds(0, E)], sbuf, sem).wait()
        pltpu.make_async_copy(zsh.at[pl.ds(0, E)], dbuf, sem).wait()

    cols = [lane + (16 * q) for q in range(W // L)]
    rowzero = jnp.zeros((L,), jnp.int32)

    def compute_stream(k, sbuf, dbuf):
        @plsc.parallel_loop(0, E, 1, unroll=2)
        def _(e):
            base = jnp.full((L,), e * W, jnp.int32)
            acc = None
            for q in range(W // L):
                idx = base + cols[q]
                sv = plsc.bitcast(plsc.load_gather(sbuf, [rowzero, idx]),
                                  jnp.bfloat16)
                dv = plsc.bitcast(plsc.load_gather(dbuf, [rowzero, idx]),
                                  jnp.bfloat16)
                p = sv * dv
                acc = p if acc is None else acc + p
            lo, hi = plsc.unpack(acc, format=plsc.PackFormat.INTERLEAVED)
            tot = plsc.cumsum(lo.astype(jnp.float32) + hi.astype(jnp.float32))
            plsc.store_scatter(out_v, [jnp.full((L,), k * E, jnp.int32) + e],
                               tot, mask=m15)

    offs = [jnp.full((L,), j * N, jnp.int32) for j in range(FW)]

    def issue_fidx(k, fsidx, fdidx, sem):
        pltpu.async_copy(src_hbm.at[pl.ds(fbase + k * CF, CF)], fsidx, sem)
        pltpu.async_copy(dst_hbm.at[pl.ds(fbase + k * CF, CF)], fdidx, sem)

    def wait_fidx(fsidx, fdidx, sem):
        pltpu.make_async_copy(src_hbm.at[pl.ds(0, CF)], fsidx, sem).wait()
        pltpu.make_async_copy(src_hbm.at[pl.ds(0, CF)], fdidx, sem).wait()

    def drain_add(part, sem):
        pltpu.make_async_copy(part, acc_sh.at[pl.ds(0, CF)], sem).wait()

    def compute_feature(k, fsidx, fdidx, part, ramp, sem_p):
        kbase = jnp.full((L,), k * CF, jnp.int32) + lane

        @plsc.parallel_loop(0, CF, L, unroll=2)
        def _(g):
            sn = fsidx[pl.ds(g, L)]
            dn = fdidx[pl.ds(g, L)]
            ramp[pl.ds(g, L)] = kbase + g
            facc = None
            for j in range(FW):
                sv = plsc.bitcast(plsc.load_gather(zcol, [sn + offs[j]]),
                                  jnp.bfloat16)
                dv = plsc.bitcast(plsc.load_gather(zcol, [dn + offs[j]]),
                                  jnp.bfloat16)
                p = sv * dv
                facc = p if facc is None else facc + p
            lo, hi = plsc.unpack(facc, format=plsc.PackFormat.INTERLEAVED)
            part[pl.ds(g, L)] = lo.astype(jnp.float32) + hi.astype(jnp.float32)

        pltpu.async_copy(part, acc_sh.at[ramp], sem_p, add=True)

    issue(0, sbuf_a, dbuf_a, sem_a)
    issue(1, sbuf_b, dbuf_b, sem_b)
    issue_fidx(0, fsidx_a, fdidx_a, sem_fa)
    issue_fidx(1, fsidx_b, fdidx_b, sem_fb)

    def pair_body(p, _):
        ka = 2 * p
        wait(sbuf_a, dbuf_a, sem_a)
        compute_stream(ka, sbuf_a, dbuf_a)

        @pl.when(p < NITER // 2 - 1)
        def _():
            issue(ka + 2, sbuf_a, dbuf_a, sem_a)

        wait_fidx(fsidx_a, fdidx_a, sem_fa)

        @pl.when(p > 0)
        def _():
            drain_add(part_a, sem_pa)

        compute_feature(ka, fsidx_a, fdidx_a, part_a, ramp_a, sem_pa)

        @pl.when(p < NITER // 2 - 1)
        def _():
            issue_fidx(ka + 2, fsidx_a, fdidx_a, sem_fa)

        wait(sbuf_b, dbuf_b, sem_b)
        compute_stream(ka + 1, sbuf_b, dbuf_b)

        @pl.when(p < NITER // 2 - 1)
        def _():
            issue(ka + 3, sbuf_b, dbuf_b, sem_b)

        wait_fidx(fsidx_b, fdidx_b, sem_fb)

        @pl.when(p > 0)
        def _():
            drain_add(part_b, sem_pb)

        compute_feature(ka + 1, fsidx_b, fdidx_b, part_b, ramp_b, sem_pb)

        @pl.when(p < NITER // 2 - 1)
        def _():
            issue_fidx(ka + 3, fsidx_b, fdidx_b, sem_fb)

        return 0

    lax.fori_loop(0, NITER // 2, pair_body, 0)
    drain_add(part_a, sem_pa)
    drain_add(part_b, sem_pb)
    plsc.subcore_barrier()

    pltpu.sync_copy(out_v, out_hbm.at[pl.ds(sbase, Y_W)])
    pltpu.sync_copy(acc_sh.at[pl.ds(s * ACC_W, ACC_W)],
                    out_hbm.at[pl.ds(fbase + s * ACC_W, ACC_W)])


@jax.jit
def kernel(z, edge_index):
    zb = z.astype(jnp.bfloat16)
    zp = lax.bitcast_convert_type(zb.reshape(N, W, 2), jnp.int32)
    zpt = zp.T.reshape(W * N)
    src = edge_index[0].astype(jnp.int32)
    dst = edge_index[1].astype(jnp.int32)
    mesh = plsc.VectorSubcoreMesh(core_axis_name="c", subcore_axis_name="s")
    run = functools.partial(
        pl.kernel,
        mesh=mesh,
        out_type=jax.ShapeDtypeStruct((B,), jnp.float32),
        compiler_params=pltpu.CompilerParams(
            use_tc_tiling_on_sc=False, needs_layout_passes=False),
        scratch_types=[
            pltpu.VMEM_SHARED((N, W), jnp.int32),
            pltpu.VMEM_SHARED((X_SC,), jnp.float32),
            pltpu.VMEM((FW * N,), jnp.int32),
            pltpu.VMEM((Y_W,), jnp.int32),
            pltpu.VMEM((Y_W,), jnp.int32),
            pltpu.VMEM((E, W), jnp.int32),
            pltpu.VMEM((E, W), jnp.int32),
            pltpu.VMEM((E, W), jnp.int32),
            pltpu.VMEM((E, W), jnp.int32),
            pltpu.VMEM((Y_W,), jnp.float32),
            pltpu.VMEM((CF,), jnp.int32),
            pltpu.VMEM((CF,), jnp.int32),
            pltpu.VMEM((CF,), jnp.int32),
            pltpu.VMEM((CF,), jnp.int32),
            pltpu.VMEM((CF,), jnp.float32),
            pltpu.VMEM((CF,), jnp.float32),
            pltpu.VMEM((CF,), jnp.int32),
            pltpu.VMEM((CF,), jnp.int32),
            pltpu.SemaphoreType.DMA,
            pltpu.SemaphoreType.DMA,
            pltpu.SemaphoreType.DMA,
            pltpu.SemaphoreType.DMA,
            pltpu.SemaphoreType.DMA,
            pltpu.SemaphoreType.DMA,
            pltpu.SemaphoreType.DMA,
        ],
    )(_edge_dot_kernel)
    return run(zp, zpt, src, dst)
